# baseline (speedup 1.0000x reference)
import jax
import jax.numpy as jnp
from jax import lax
from jax.experimental import pallas as pl
from jax.experimental.pallas import tpu as pltpu
from jax.experimental.pallas import tpu_sc as plsc

VOCAB = 1_000_000
D = 64
B = 4096 * 200
NC, NS = 2, 16
NW = NC * NS
BPW = B // NW
C = 128
CHUNKS = BPW // C


def _body(x_hbm, table_hbm, out_hbm, idx_v, rows_v, sem):
    c = lax.axis_index("c")
    s = lax.axis_index("s")
    wid = s * NC + c
    pltpu.sync_copy(x_hbm.at[wid], idx_v)
    base = wid * BPW

    def chunk(i, carry):
        pltpu.async_copy(table_hbm.at[idx_v.at[i]], rows_v, sem).wait()
        pltpu.sync_copy(rows_v, out_hbm.at[pl.ds(base + i * C, C)])
        return carry

    lax.fori_loop(0, CHUNKS, chunk, 0)


@jax.jit
def kernel(x, table):
    xr = x.reshape(NW, CHUNKS, C).astype(jnp.int32)
    mesh = plsc.VectorSubcoreMesh(
        core_axis_name="c", subcore_axis_name="s", num_cores=NC, num_subcores=NS
    )
    out = pl.kernel(
        _body,
        out_type=jax.ShapeDtypeStruct((B, D), jnp.float32),
        mesh=mesh,
        scratch_types=[
            pltpu.VMEM((CHUNKS, C), jnp.int32),
            pltpu.VMEM((C, D), jnp.float32),
            pltpu.SemaphoreType.DMA,
        ],
        compiler_params=pltpu.CompilerParams(use_tc_tiling_on_sc=False),
    )(xr, table)
    return out.reshape(x.shape[0], x.shape[1], D)


# speedup vs baseline: 1.1199x; 1.1199x over previous
import jax
import jax.numpy as jnp
from jax import lax
from jax.experimental import pallas as pl
from jax.experimental.pallas import tpu as pltpu
from jax.experimental.pallas import tpu_sc as plsc

VOCAB = 1_000_000
D = 64
B = 4096 * 200
NC, NS = 2, 16
NW = NC * NS
BPW = B // NW
C = 128
CHUNKS = BPW // C


NBUF = 4
GROUPS = CHUNKS // NBUF


def _body(x_hbm, table_hbm, out_hbm, idx_v, rows_v, gsem, osem):
    c = lax.axis_index("c")
    s = lax.axis_index("s")
    wid = s * NC + c
    pltpu.sync_copy(x_hbm.at[wid], idx_v)
    base = wid * BPW

    def gather_desc(g, bank, b):
        i = g * NBUF + b
        return pltpu.make_async_copy(
            table_hbm.at[idx_v.at[i]], rows_v.at[bank, b], gsem.at[bank, b]
        )

    def write_desc(g, bank, b):
        i = g * NBUF + b
        return pltpu.make_async_copy(
            rows_v.at[bank, b], out_hbm.at[pl.ds(base + i * C, C)], osem.at[bank, b]
        )

    for b in range(NBUF):
        gather_desc(0, 0, b).start()

    def pair(p, carry):
        for h in range(2):
            g = 2 * p + h
            bank = h
            for b in range(NBUF):

                @pl.when(g + 1 < GROUPS)
                def _():
                    @pl.when(g >= 1)
                    def _():
                        write_desc(g - 1, 1 - bank, b).wait()

                    gather_desc(g + 1, 1 - bank, b).start()

            for b in range(NBUF):
                gather_desc(g, bank, b).wait()
                write_desc(g, bank, b).start()
        return carry

    lax.fori_loop(0, GROUPS // 2, pair, 0)
    for b in range(NBUF):
        write_desc(GROUPS - 2, 0, b).wait()
        write_desc(GROUPS - 1, 1, b).wait()


@jax.jit
def kernel(x, table):
    xr = x.reshape(NW, CHUNKS, C).astype(jnp.int32)
    mesh = plsc.VectorSubcoreMesh(
        core_axis_name="c", subcore_axis_name="s", num_cores=NC, num_subcores=NS
    )
    out = pl.kernel(
        _body,
        out_type=jax.ShapeDtypeStruct((B, D), jnp.float32),
        mesh=mesh,
        scratch_types=[
            pltpu.VMEM((CHUNKS, C), jnp.int32),
            pltpu.VMEM((2, NBUF, C, D), jnp.float32),
            pltpu.SemaphoreType.DMA((2, NBUF)),
            pltpu.SemaphoreType.DMA((2, NBUF)),
        ],
        compiler_params=pltpu.CompilerParams(use_tc_tiling_on_sc=False),
    )(xr, table)
    return out.reshape(x.shape[0], x.shape[1], D)
